# Initial kernel scaffold; baseline (speedup 1.0000x reference)
import jax
import jax.numpy as jnp
from jax.experimental import pallas as pl


def kernel(crd, probs, params, nb_it):
    raise NotImplementedError("write your pallas kernel here")


# speedup vs baseline: 7.1503x; 7.1503x over previous
import functools

import jax
import jax.numpy as jnp
from jax import lax
from jax.experimental import pallas as pl
from jax.experimental.pallas import tpu as pltpu
from jax.experimental.pallas import tpu_sc as plsc

N = 1024
K = 48
HEADS = 8
SKD = 16
SVD = 16
PKD = 4
PVD = 4
PAIR = 32
NB_AA = 20
EMBED = 256

_HI = lax.Precision.HIGHEST
_DF = lax.Precision.DEFAULT
_S1 = float((3 * SKD) ** -0.5)
_S2 = float(((3 * PKD) * 4.5) ** -0.5)
_S3 = float(3.0 ** -0.5)


def _dotT(a, b, prec=_DF):
    return lax.dot_general(a, b, (((1,), (1,)), ((), ())), precision=prec)


def _dot(a, b, prec=_HI):
    return jnp.dot(a, b, precision=prec)


def _dotd(a, b):
    return jnp.dot(a, b, precision=_DF)


def _bf(v):
    return v.astype(jnp.bfloat16).astype(jnp.float32)


_BN_A = 128


def _frames_knn_body(crd_full_ref, caT_blk_ref, blk_ref, R_ref, crot_ref,
                     idx_ref, d2_ref):
    blk = blk_ref[...]
    ca = blk[:, 3:6]
    b1 = blk[:, 6:9] - ca
    b1 = b1 / jnp.sqrt(jnp.sum(b1 * b1, axis=1, keepdims=True))
    c0 = blk[:, 0:3] - ca

    def cross(u, v):
        return jnp.concatenate([
            u[:, 1:2] * v[:, 2:3] - u[:, 2:3] * v[:, 1:2],
            u[:, 2:3] * v[:, 0:1] - u[:, 0:1] * v[:, 2:3],
            u[:, 0:1] * v[:, 1:2] - u[:, 1:2] * v[:, 0:1],
        ], axis=1)

    b2 = cross(b1, c0)
    b2 = b2 / jnp.sqrt(jnp.sum(b2 * b2, axis=1, keepdims=True))
    b3 = cross(b1, b2)
    R9 = jnp.concatenate([b1, b2, b3], axis=1)
    R_ref[...] = R9

    cm = blk - jnp.concatenate([ca, ca, ca, ca], axis=1)
    cols = []
    for a in range(4):
        for j in range(3):
            cols.append(
                R9[:, 3 * j:3 * j + 1] * cm[:, 3 * a:3 * a + 1]
                + R9[:, 3 * j + 1:3 * j + 2] * cm[:, 3 * a + 1:3 * a + 2]
                + R9[:, 3 * j + 2:3 * j + 3] * cm[:, 3 * a + 2:3 * a + 3])
    crot_ref[...] = jnp.concatenate(cols, axis=1)

    ca_all = crd_full_ref[...][:, 3:6]
    caT_blk = caT_blk_ref[...]
    d2 = jnp.zeros((N, _BN_A), jnp.float32)
    for c in range(3):
        d = ca_all[:, c:c + 1] - caT_blk[c:c + 1, :]
        d2 = d2 + d * d
    d2_ref[...] = d2

    iota = lax.broadcasted_iota(jnp.int32, (N, _BN_A), 0)
    big = jnp.int32(1 << 30)

    def body(k, _):
        d = d2_ref[...]
        mn = jnp.min(d, axis=0, keepdims=True)
        cand = jnp.where(d == mn, iota, big)
        mi = jnp.min(cand, axis=0, keepdims=True)
        idx_ref[pl.ds(k, 1), :] = mi
        d2_ref[...] = jnp.where(cand == mi, jnp.float32(jnp.inf), d)
        return 0

    lax.fori_loop(0, K, body, 0)


def _frames_knn(crd12, caT):
    grid = N // _BN_A
    return pl.pallas_call(
        _frames_knn_body,
        grid=(grid,),
        in_specs=[
            pl.BlockSpec((N, 12), lambda i: (0, 0)),
            pl.BlockSpec((3, _BN_A), lambda i: (0, i)),
            pl.BlockSpec((_BN_A, 12), lambda i: (i, 0)),
        ],
        out_specs=[
            pl.BlockSpec((_BN_A, 9), lambda i: (i, 0)),
            pl.BlockSpec((_BN_A, 12), lambda i: (i, 0)),
            pl.BlockSpec((K, _BN_A), lambda i: (0, i)),
        ],
        out_shape=[
            jax.ShapeDtypeStruct((N, 9), jnp.float32),
            jax.ShapeDtypeStruct((N, 12), jnp.float32),
            jax.ShapeDtypeStruct((K, N), jnp.int32),
        ],
        scratch_shapes=[pltpu.VMEM((N, _BN_A), jnp.float32)],
    )(crd12, caT, crd12)


def _gather_rows(table, idx_flat):
    B = idx_flat.shape[0]
    D = table.shape[1]
    NW = 32
    CH = 128
    per_w = B // NW
    nch = per_w // CH
    mesh = plsc.VectorSubcoreMesh(core_axis_name="c", subcore_axis_name="s")

    @functools.partial(
        pl.kernel,
        mesh=mesh,
        out_type=jax.ShapeDtypeStruct((B, D), jnp.float32),
        scratch_types=[
            pltpu.VMEM((CH,), jnp.int32),
            pltpu.VMEM((CH, D), jnp.float32),
            pltpu.SemaphoreType.DMA,
        ],
    )
    def k(table_hbm, idx_hbm, out_hbm, idx_v, rows_v, sem):
        wid = lax.axis_index("s") * 2 + lax.axis_index("c")
        base = wid * per_w

        def body(c, _):
            off = base + c * CH
            pltpu.sync_copy(idx_hbm.at[pl.ds(off, CH)], idx_v)
            pltpu.async_copy(table_hbm.at[idx_v], rows_v, sem).wait()
            pltpu.sync_copy(rows_v, out_hbm.at[pl.ds(off, CH)])
            return 0

        lax.fori_loop(0, nch, body, 0)

    return k(table, idx_flat)


def _resmlp_apply(x, ws, nblocks):
    h = jax.nn.relu(_dotd(x, ws[0]) + ws[1])
    p = 2
    for _ in range(nblocks):
        r = h
        for _ in range(2):
            h = jax.nn.relu(_dotd(h, ws[p]) + ws[p + 1])
            p += 2
        h = h + r
    return _dotd(h, ws[p]) + ws[p + 1]


def _input_resnet_body(*refs):
    crot_ref, probs_ref = refs[0], refs[1]
    wrefs = refs[2:-1]
    out_ref = refs[-1]
    x = jnp.concatenate([crot_ref[...], probs_ref[...]], axis=1)
    ws = [r[...] for r in wrefs]
    out_ref[...] = _resmlp_apply(x, ws, 2)


def _input_resnet(crot, probs, ws):
    return pl.pallas_call(
        _input_resnet_body,
        grid=(1,),
        in_specs=[pl.BlockSpec(a.shape, lambda i, n=len(a.shape): (0,) * n)
                  for a in [crot, probs] + ws],
        out_specs=pl.BlockSpec((N, EMBED), lambda i: (0, 0)),
        out_shape=jax.ShapeDtypeStruct((N, EMBED), jnp.float32),
    )(crot, probs, *ws)


_BN_D = 512


def _onehot(rows, cols, fn):
    rr = lax.broadcasted_iota(jnp.int32, (rows, cols), 0)
    cc = lax.broadcasted_iota(jnp.int32, (rows, cols), 1)
    return jnp.where(fn(rr, cc), 1.0, 0.0).astype(jnp.float32)


def _pairwise_body(*refs):
    g_ref, loc_ref = refs[0], refs[1]
    wrefs = refs[2:-1]
    out_ref = refs[-1]
    g = g_ref[...]
    loc = loc_ref[...]

    Csel = _onehot(128, 12, lambda r, c: r == c)
    Dsel = _onehot(44, 12, lambda r, c: r == 29 + c % 3)
    cm = _dot(g, Csel) - _dot(loc, Dsel)
    acc = jnp.zeros_like(cm)
    for i in range(3):
        Ai = _onehot(44, 12, lambda r, c: r == 20 + 3 * (c % 3) + i)
        Bi = _onehot(12, 12, lambda r, c: r == 3 * (c // 3) + i)
        acc = acc + _dot(loc, Ai) * _dot(cm, Bi)
    Pps = _onehot(44, 64, lambda r, c: (c < 20) & (r == c))
    Ppn = _onehot(128, 64, lambda r, c: (c >= 20) & (c < 40) & (r == c - 8))
    Pcpr = _onehot(12, 64, lambda r, c: (c >= 40) & (c < 52) & (r == c - 40))
    Pcrot = _onehot(44, 64, lambda r, c: (c >= 52) & (r == c - 20))
    x = _dot(loc, Pps) + _dot(g, Ppn) + _dot(acc, Pcpr) + _dot(loc, Pcrot)
    ws = [r[...] for r in wrefs]
    out_ref[...] = _resmlp_apply(x, ws, 2)


def _pairwise_mlp(g1, loc, ws):
    B = g1.shape[0]
    grid = B // _BN_D
    return pl.pallas_call(
        _pairwise_body,
        grid=(grid,),
        in_specs=[pl.BlockSpec((_BN_D, 128), lambda i: (i, 0)),
                  pl.BlockSpec((_BN_D, 44), lambda i: (i, 0))] +
                 [pl.BlockSpec(a.shape, lambda i, n=len(a.shape): (0,) * n)
                  for a in ws],
        out_specs=pl.BlockSpec((_BN_D, PAIR), lambda i: (i, 0)),
        out_shape=jax.ShapeDtypeStruct((B, PAIR), jnp.float32),
    )(g1, loc, *ws)


def _proj_body(x_ref, R_ref, ca_ref, wq_ref, wk_ref, wv_ref, wqp_ref, wkp_ref,
               wvp_ref, perm_ref, m968_ref,
               qs_ref, ks_ref, vs_ref, qpg_ref, kpg_ref, vpg_ref,
               q2_ref, k2_ref):
    x = x_ref[...]
    R9 = R_ref[...]
    ca = ca_ref[...]
    perm = perm_ref[...]
    m968 = m968_ref[...]

    qs_ref[...] = _dotd(x, wq_ref[...])
    ks_ref[...] = _dotd(x, wk_ref[...])
    vs_ref[...] = _dotd(x, wv_ref[...])

    def rot(w_cg):
        p_cg = _dotd(x, w_cg)
        groups = []
        for r in range(3):
            acc = ca[:, r:r + 1] * jnp.ones((1, 32), jnp.float32)
            for c in range(3):
                acc = acc + (p_cg[:, c * 32:(c + 1) * 32]
                             * R9[:, 3 * c + r:3 * c + r + 1])
            groups.append(acc)
        p_rm = jnp.concatenate(groups, axis=1)
        return _dot(p_rm, perm)

    qpg = rot(wqp_ref[...])
    kpg = rot(wkp_ref[...])
    vpg = rot(wvp_ref[...])
    qpg_ref[...] = qpg
    kpg_ref[...] = kpg
    vpg_ref[...] = vpg
    q2_ref[...] = _dot(qpg * qpg, m968)
    k2_ref[...] = _dot(kpg * kpg, m968)


def _projections(x, R9, ca, wq, wk, wv, wqp_cg, wkp_cg, wvp_cg, perm, m968):
    outs = [
        jax.ShapeDtypeStruct((N, HEADS * SKD), jnp.float32),
        jax.ShapeDtypeStruct((N, HEADS * SKD), jnp.float32),
        jax.ShapeDtypeStruct((N, HEADS * SVD), jnp.float32),
        jax.ShapeDtypeStruct((N, 96), jnp.float32),
        jax.ShapeDtypeStruct((N, 96), jnp.float32),
        jax.ShapeDtypeStruct((N, 96), jnp.float32),
        jax.ShapeDtypeStruct((N, 8), jnp.float32),
        jax.ShapeDtypeStruct((N, 8), jnp.float32),
    ]
    args = [x, R9, ca, wq, wk, wv, wqp_cg, wkp_cg, wvp_cg, perm, m968]
    return pl.pallas_call(
        _proj_body,
        grid=(1,),
        in_specs=[pl.BlockSpec(a.shape, lambda i, n=len(a.shape): (0,) * n)
                  for a in args],
        out_specs=[pl.BlockSpec(o.shape, lambda i: (0, 0)) for o in outs],
        out_shape=outs,
    )(*args)


_BN_F = 256


def _attn_body(qs_ref, qpg_ref, q2_ref, ks_ref, kpg_ref, vs_ref, vpg_ref,
               k2T_ref, pw_ref, m_ref, z_ref, sv_ref, spv_ref):
    qs = qs_ref[...]
    qpg = qpg_ref[...]
    q2 = q2_ref[...]
    ks = ks_ref[...]
    kpg = kpg_ref[...]
    vs = vs_ref[...]
    vpg = vpg_ref[...]
    k2T = k2T_ref[...]
    pw = pw_ref[...]
    ms, zs, svs, spvs = [], [], [], []
    for h in range(HEADS):
        qh = qs[:, h * SKD:(h + 1) * SKD]
        kh = ks[:, h * SKD:(h + 1) * SKD]
        qph = qpg[:, h * 12:(h + 1) * 12]
        kph = kpg[:, h * 12:(h + 1) * 12]
        vh = vs[:, h * SVD:(h + 1) * SVD]
        vph = vpg[:, h * 12:(h + 1) * 12]
        qk = _dotT(qh, kh)
        dist = (q2[:, h:h + 1] + k2T[h:h + 1, :]) - 2.0 * _dotT(qph, kph)
        logit = qk * _S1 - (0.5 * _S2) * pw[0, h] * dist
        m = jnp.max(logit, axis=1, keepdims=True)
        e = jnp.exp(logit - m)
        ms.append(m)
        zs.append(jnp.sum(e, axis=1, keepdims=True))
        svs.append(_dotd(e, vh))
        spvs.append(_dotd(e, vph))
    m_ref[...] = jnp.concatenate(ms, axis=1)
    z_ref[...] = jnp.concatenate(zs, axis=1)
    sv_ref[...] = jnp.concatenate(svs, axis=1)
    spv_ref[...] = jnp.concatenate(spvs, axis=1)


def _dense_attn(qs, qpg, q2, ks, kpg, vs, vpg, k2T, pw):
    grid = N // _BN_F
    blk = lambda w: pl.BlockSpec((_BN_F, w), lambda i: (i, 0))
    full = lambda a: pl.BlockSpec(a.shape, lambda i: tuple(0 for _ in a.shape))
    outs = [
        jax.ShapeDtypeStruct((N, 8), jnp.float32),
        jax.ShapeDtypeStruct((N, 8), jnp.float32),
        jax.ShapeDtypeStruct((N, 128), jnp.float32),
        jax.ShapeDtypeStruct((N, 96), jnp.float32),
    ]
    return pl.pallas_call(
        _attn_body,
        grid=(grid,),
        in_specs=[blk(128), blk(96), blk(8), full(ks), full(kpg), full(vs),
                  full(vpg), full(k2T), full(pw)],
        out_specs=[blk(8), blk(8), blk(128), blk(96)],
        out_shape=outs,
    )(qs, qpg, q2, ks, kpg, vs, vpg, k2T, pw)


_BN_G = 32


def _tail_body(*refs):
    (kt_ref, neigh_ref, qs_ref, qpg_ref, q2_ref, m_ref, z_ref, sv_ref,
     spv_ref, x_ref, R_ref, ca_ref, wpair_ref, pw_ref) = refs[:14]
    wrefs = refs[14:-2]
    pout_ref, conf_ref = refs[-2], refs[-1]

    B = _BN_G * K
    kt = kt_ref[...]
    neigh = neigh_ref[...]
    pw = pw_ref[...]
    onehot = _onehot

    def expand(v):
        w = v.shape[1]
        return jnp.broadcast_to(v[:, None, :], (_BN_G, K, w)).reshape(B, w)

    def segsum(u):
        return u.reshape(_BN_G, K, u.shape[1]).sum(axis=1)

    M1 = onehot(128, 8, lambda r, c: r // 16 == c)
    M12 = onehot(96, 8, lambda r, c: r // 12 == c)
    T1 = onehot(8, 128, lambda r, c: c // 16 == r)
    T12 = onehot(8, 96, lambda r, c: c // 12 == r)
    E3 = onehot(3, 96, lambda r, c: c % 3 == r)
    M3q = onehot(96, 32, lambda r, c: r // 3 == c)

    qs_e = expand(qs_ref[...])
    qpg_e = expand(qpg_ref[...])
    q2_e = expand(q2_ref[...])
    m_e = expand(m_ref[...])

    ks_g = kt[:, 0:128]
    k2_g = kt[:, 128:136]
    kpg_g = kt[:, 136:232]
    vs_g = kt[:, 256:384]
    vpg_g = kt[:, 384:480]

    qk = _dot(_bf(qs_e) * _bf(ks_g), M1)
    qk_pt = _dot(_bf(qpg_e) * _bf(kpg_g), M12)
    dist = q2_e + k2_g - 2.0 * qk_pt
    base = qk * _S1 - (0.5 * _S2) * pw * dist
    delta = _dotd(neigh, wpair_ref[...]) * _S3
    eb = jnp.exp(base - m_e)
    ed = jnp.exp(base + delta - m_e)
    corr = ed - eb

    Zf = z_ref[...] + segsum(corr)
    Svc = segsum(_dot(corr, T1) * _bf(vs_g))
    Spvc = segsum(_dot(corr, T12) * _bf(vpg_g))
    attn_nb = ed / expand(Zf)
    X1 = onehot(8, 256, lambda r, c: c // 32 == r)
    X2 = onehot(32, 256, lambda r, c: c % 32 == r)
    opair = segsum(_dotd(attn_nb, X1) * _dotd(neigh, X2))

    Zinv = 1.0 / Zf
    os_ = (sv_ref[...] + Svc) * _dot(Zinv, T1)
    opg = (spv_ref[...] + Spvc) * _dot(Zinv, T12)
    ca = ca_ref[...]
    R9 = R_ref[...]
    opgm = opg - _dot(ca, E3)
    op_hdr = jnp.zeros((_BN_G, 96), jnp.float32)
    for r in range(3):
        Qr = onehot(32, 96, lambda rr, cc: (cc // 3 == rr) & (cc % 3 == r))
        acc = jnp.zeros((_BN_G, 32), jnp.float32)
        for c in range(3):
            Pc = onehot(96, 32, lambda rr, cc: (rr // 3 == cc) & (rr % 3 == c))
            acc = acc + _dot(opgm, Pc) * R9[:, 3 * r + c:3 * r + c + 1]
        op_hdr = op_hdr + _dot(acc, Qr)
    onorm = jnp.sqrt(_dot(op_hdr * op_hdr, M3q) + 1e-8)
    feat = jnp.concatenate([os_, opair, op_hdr, onorm], axis=1)

    ws = [r[...] for r in wrefs]
    (wo, bo, g1, b1g, wf1, bf1, wf2, bf2, wf3, bf3, g2, b2g) = ws[:12]
    ors = ws[12:]

    x = x_ref[...] + _dotd(feat, wo) + bo

    def ln(v, g, b):
        mu = jnp.mean(v, axis=1, keepdims=True)
        var = jnp.mean((v - mu) * (v - mu), axis=1, keepdims=True)
        return (v - mu) / jnp.sqrt(var + 1e-5) * g + b

    x = ln(x, g1, b1g)
    hdd = jax.nn.relu(_dotd(x, wf1) + bf1)
    hdd = jax.nn.relu(_dotd(hdd, wf2) + bf2)
    hdd = _dotd(hdd, wf3) + bf3
    ft = ln(x + hdd, g2, b2g)

    out = _resmlp_apply(ft, ors, 1)
    logits = out[:, 0:NB_AA]
    mx = jnp.max(logits, axis=1, keepdims=True)
    ee = jnp.exp(logits - mx)
    po = ee / jnp.sum(ee, axis=1, keepdims=True)
    conf = 1.0 / (1.0 + jnp.exp(-out[:, NB_AA:NB_AA + 1]))
    pout_ref[...] = conf * po + (1.0 - conf) / NB_AA
    conf_ref[...] = conf


def _tail(kt, neigh, qs, qpg, q2, m, z, sv, spv, x, R9, ca, wpair, pw, ws):
    grid = N // _BN_G
    Bb = _BN_G * K
    nblk = lambda w: pl.BlockSpec((_BN_G, w), lambda i: (i, 0))
    jblk = lambda w: pl.BlockSpec((Bb, w), lambda i: (i, 0))
    full = lambda a: pl.BlockSpec(a.shape, lambda i: tuple(0 for _ in a.shape))
    args = [kt, neigh, qs, qpg, q2, m, z, sv, spv, x, R9, ca, wpair, pw] + ws
    specs = ([jblk(kt.shape[1]), jblk(PAIR), nblk(128), nblk(96), nblk(8),
              nblk(8), nblk(8), nblk(128), nblk(96), nblk(EMBED), nblk(9),
              nblk(3), full(wpair), full(pw)] + [full(a) for a in ws])
    outs = [jax.ShapeDtypeStruct((N, NB_AA), jnp.float32),
            jax.ShapeDtypeStruct((N, 1), jnp.float32)]
    return pl.pallas_call(
        _tail_body,
        grid=(grid,),
        in_specs=specs,
        out_specs=[nblk(NB_AA), nblk(1)],
        out_shape=outs,
    )(*args)


def _mlp_ws(p):
    ws = [p['in']['W'], p['in']['b'].reshape(1, -1)]
    for blk in p['blocks']:
        for l in blk:
            ws += [l['W'], l['b'].reshape(1, -1)]
    ws += [p['out']['W'], p['out']['b'].reshape(1, -1)]
    return ws


def kernel(crd, probs, params, nb_it):
    crd12 = crd.reshape(N, 12)
    ca = crd[:, 1, :]
    caT = ca.T

    R9, crot, idxT = _frames_knn(crd12, caT)
    idx_flat = idxT.T.reshape(N * K)

    tbl1 = jnp.concatenate(
        [crd12, probs, jnp.zeros((N, 96), jnp.float32)], axis=1)
    g1 = _gather_rows(tbl1, idx_flat)

    x = _input_resnet(crot, probs, _mlp_ws(params['input_resnet']))

    loc = jnp.concatenate([probs, R9, ca, crot], axis=1)
    loc_e = jnp.broadcast_to(loc[:, None, :], (N, K, 44)).reshape(N * K, 44)
    neigh = _pairwise_mlp(g1, loc_e, _mlp_ws(params['pairwise_resnet']))

    tp = params['trans'][0]
    def cg(w):
        w3 = w.reshape(EMBED, HEADS * PKD, 3)
        return jnp.transpose(w3, (0, 2, 1)).reshape(EMBED, 96)

    src = jnp.arange(96)
    r_, hd = src // 32, src % 32
    dst = (hd // PKD) * 12 + (hd % PKD) * 3 + r_
    perm = jnp.zeros((96, 96), jnp.float32).at[src, dst].set(1.0)
    m968 = (jnp.arange(96)[:, None] // 12 ==
            jnp.arange(8)[None, :]).astype(jnp.float32)

    qs, ks, vs, qpg, kpg, vpg, q2, k2 = _projections(
        x, R9, ca, tp['q']['W'], tp['k']['W'], tp['v']['W'],
        cg(tp['qp']['W']), cg(tp['kp']['W']), cg(tp['vp']['W']), perm, m968)

    pad24 = jnp.zeros((N, 24), jnp.float32)
    pad32 = jnp.zeros((N, 32), jnp.float32)
    tbl2 = jnp.concatenate([ks, k2, kpg, pad24, vs, vpg, pad32],
                           axis=1)
    kt = _gather_rows(tbl2, idx_flat)

    pw = jax.nn.softplus(tp['pw']).reshape(1, HEADS)
    m, z, sv, spv = _dense_attn(qs, qpg, q2, ks, kpg, vs, vpg, k2.T, pw)

    tail_ws = [tp['out']['W'], tp['out']['b'].reshape(1, -1),
               tp['ln1']['g'].reshape(1, -1), tp['ln1']['b'].reshape(1, -1),
               tp['ff1']['W'], tp['ff1']['b'].reshape(1, -1),
               tp['ff2']['W'], tp['ff2']['b'].reshape(1, -1),
               tp['ff3']['W'], tp['ff3']['b'].reshape(1, -1),
               tp['ln2']['g'].reshape(1, -1), tp['ln2']['b'].reshape(1, -1),
               ] + _mlp_ws(params['output_resnet'])
    pout, conf = _tail(kt, neigh, qs, qpg, q2, m, z, sv, spv, x, R9, ca,
                       tp['pair']['W'], pw, tail_ws)
    return pout.reshape(1, N, NB_AA), conf.reshape(1, N)
